# baseline (speedup 1.0000x reference)
import jax
import jax.numpy as jnp
from jax import lax
from jax.experimental import pallas as pl
from jax.experimental.pallas import tpu as pltpu
from jax.experimental.pallas import tpu_sc as plsc

VOCAB = 1000000
EMBED_DIM = 64
BATCH = 4096
HIST_LEN = 200

NC = 2
NS = 16
NW = NC * NS
L = 16

NPAIR = VOCAB // 2
NBUF = 4
SBUF = 2


def _gather_body(idx_hbm, tbl_hbm, out_hbm, slab_v, ibuf_v, pofs_v, g_v,
                 sel_v, gsems, ssems):
    w = lax.axis_index("s") * NC + lax.axis_index("c")
    iota = lax.iota(jnp.int32, L)
    dchunk = [iota + L * c for c in range(4)]

    pltpu.sync_copy(idx_hbm.at[:, pl.ds(128 * w, 128)], slab_v)

    def build(t, b):
        for m in range(8):
            v = slab_v[t, pl.ds(L * m, L)]
            ibuf_v[b, pl.ds(L * m, L)] = lax.shift_right_logical(v, 1)
            pofs_v[b, pl.ds(L * m, L)] = lax.shift_left(
                lax.bitwise_and(v, 1), 6)

    def fire(t, b):
        build(t, b)
        pltpu.async_copy(tbl_hbm.at[ibuf_v.at[b]], g_v.at[b], gsems.at[b])

    def gwait(b):
        pltpu.make_async_copy(tbl_hbm.at[ibuf_v.at[b]], g_v.at[b],
                              gsems.at[b]).wait()

    def select(b, s):
        def row(l, _):
            lsp = lax.broadcast(l, (L,))
            base = plsc.load_gather(pofs_v.at[b], [lsp]) + iota
            for c in range(4):
                vals = plsc.load_gather(g_v.at[b, l], [base + L * c])
                plsc.store_scatter(sel_v.at[s], [dchunk[c], lsp], vals)
            return ()
        lax.fori_loop(0, 128, row, (), unroll=4)

    def store(t, s):
        pltpu.async_copy(sel_v.at[s],
                         out_hbm.at[t, :, pl.ds(128 * w, 128)], ssems.at[s])

    def swait(t, s):
        pltpu.make_async_copy(sel_v.at[s],
                              out_hbm.at[t, :, pl.ds(128 * w, 128)],
                              ssems.at[s]).wait()

    def step(t, b, s, prev_store, next_fire):
        gwait(b)
        if prev_store:
            swait(t - SBUF, s)
        select(b, s)
        store(t, s)
        if next_fire:
            fire(t + NBUF, b)

    for b in range(NBUF):
        fire(b, b)
    for t in range(NBUF):
        step(t, t, t % SBUF, t >= SBUF, True)

    def body(gi, _):
        for b in range(NBUF):
            step(NBUF * gi + b, b, b % SBUF, True, True)
        return ()

    lax.fori_loop(1, HIST_LEN // NBUF - 1, body, (), unroll=False)

    for b in range(NBUF):
        step(HIST_LEN - NBUF + b, b, b % SBUF, True, False)
    for s in range(SBUF):
        swait(HIST_LEN - SBUF + s, s)


@jax.jit
def _gather(idx_t, tbl):
    mesh = plsc.VectorSubcoreMesh(core_axis_name="c", subcore_axis_name="s")
    kern = pl.kernel(
        _gather_body,
        out_type=jax.ShapeDtypeStruct((HIST_LEN, EMBED_DIM, BATCH),
                                      jnp.float32),
        mesh=mesh,
        scratch_types=[
            pltpu.VMEM((HIST_LEN, 128), jnp.int32),
            pltpu.VMEM((NBUF, 128), jnp.int32),
            pltpu.VMEM((NBUF, 128), jnp.int32),
            pltpu.VMEM((NBUF, 128, 128), jnp.float32),
            pltpu.VMEM((SBUF, EMBED_DIM, 128), jnp.float32),
            pltpu.SemaphoreType.DMA((NBUF,)),
            pltpu.SemaphoreType.DMA((SBUF,)),
        ],
        compiler_params=pltpu.CompilerParams(
            use_tc_tiling_on_sc=True, needs_layout_passes=False),
    )
    return kern(idx_t, tbl)


def kernel(inputs, embeddings):
    idx_t = inputs.astype(jnp.int32).T
    tbl = embeddings.reshape(NPAIR, 128)
    out_t = _gather(idx_t, tbl)
    return out_t.transpose(2, 0, 1)


# speedup vs baseline: 1.1256x; 1.1256x over previous
import jax
import jax.numpy as jnp
from jax import lax
from jax.experimental import pallas as pl
from jax.experimental.pallas import tpu as pltpu
from jax.experimental.pallas import tpu_sc as plsc

VOCAB = 1000000
EMBED_DIM = 64
BATCH = 4096
HIST_LEN = 200

NC = 2
NS = 16
NW = NC * NS
L = 16

NPAIR = VOCAB // 2
NBUF = 4
SBUF = 2


PITCH = 65


def _gather_body(idx_hbm, tbl_hbm, out_hbm, slab_v, ibuf_v, pofs_v, g_v,
                 p65_v, sel_v, gsems, ssems):
    w = lax.axis_index("s") * NC + lax.axis_index("c")
    iota = lax.iota(jnp.int32, L)
    l65 = [(iota + L * m) * PITCH for m in range(8)]

    pltpu.sync_copy(idx_hbm.at[:, pl.ds(128 * w, 128)], slab_v)

    def build(t, b):
        for m in range(8):
            v = slab_v[t, pl.ds(L * m, L)]
            ibuf_v[b, pl.ds(L * m, L)] = lax.shift_right_logical(v, 1)
            pofs_v[b, pl.ds(L * m, L)] = lax.shift_left(
                lax.bitwise_and(v, 1), 6)

    def fire(t, b):
        build(t, b)
        pltpu.async_copy(tbl_hbm.at[ibuf_v.at[b]], g_v.at[b], gsems.at[b])

    def gwait(b):
        pltpu.make_async_copy(tbl_hbm.at[ibuf_v.at[b]], g_v.at[b],
                              gsems.at[b]).wait()

    def select(b, s):
        def row(l, _):
            lsp = lax.broadcast(l, (L,))
            rbase = plsc.load_gather(pofs_v.at[b], [lsp]) + iota
            base = PITCH * l
            for c in range(4):
                p65_v[pl.ds(base + L * c, L)] = plsc.load_gather(
                    g_v.at[b, l], [rbase + L * c])
            return ()
        lax.fori_loop(0, 128, row, (), unroll=4)

        def drow(d, _):
            for m in range(8):
                sel_v[s, d, pl.ds(L * m, L)] = plsc.load_gather(
                    p65_v, [l65[m] + d])
            return ()
        lax.fori_loop(0, EMBED_DIM, drow, (), unroll=2)

    def store(t, s):
        pltpu.async_copy(sel_v.at[s],
                         out_hbm.at[t, :, pl.ds(128 * w, 128)], ssems.at[s])

    def swait(t, s):
        pltpu.make_async_copy(sel_v.at[s],
                              out_hbm.at[t, :, pl.ds(128 * w, 128)],
                              ssems.at[s]).wait()

    def step(t, b, s, prev_store, next_fire):
        gwait(b)
        if prev_store:
            swait(t - SBUF, s)
        select(b, s)
        store(t, s)
        if next_fire:
            fire(t + NBUF, b)

    for b in range(NBUF):
        fire(b, b)
    for t in range(NBUF):
        step(t, t, t % SBUF, t >= SBUF, True)

    def body(gi, _):
        for b in range(NBUF):
            step(NBUF * gi + b, b, b % SBUF, True, True)
        return ()

    lax.fori_loop(1, HIST_LEN // NBUF - 1, body, (), unroll=False)

    for b in range(NBUF):
        step(HIST_LEN - NBUF + b, b, b % SBUF, True, False)
    for s in range(SBUF):
        swait(HIST_LEN - SBUF + s, s)


@jax.jit
def _gather(idx_t, tbl):
    mesh = plsc.VectorSubcoreMesh(core_axis_name="c", subcore_axis_name="s")
    kern = pl.kernel(
        _gather_body,
        out_type=jax.ShapeDtypeStruct((HIST_LEN, EMBED_DIM, BATCH),
                                      jnp.float32),
        mesh=mesh,
        scratch_types=[
            pltpu.VMEM((HIST_LEN, 128), jnp.int32),
            pltpu.VMEM((NBUF, 128), jnp.int32),
            pltpu.VMEM((NBUF, 128), jnp.int32),
            pltpu.VMEM((NBUF, 128, 128), jnp.float32),
            pltpu.VMEM((PITCH * 128,), jnp.float32),
            pltpu.VMEM((SBUF, EMBED_DIM, 128), jnp.float32),
            pltpu.SemaphoreType.DMA((NBUF,)),
            pltpu.SemaphoreType.DMA((SBUF,)),
        ],
        compiler_params=pltpu.CompilerParams(
            use_tc_tiling_on_sc=True, needs_layout_passes=False),
    )
    return kern(idx_t, tbl)


def kernel(inputs, embeddings):
    idx_t = inputs.astype(jnp.int32).T
    tbl = embeddings.reshape(NPAIR, 128)
    out_t = _gather(idx_t, tbl)
    return out_t.transpose(2, 0, 1)


# speedup vs baseline: 1.1316x; 1.0053x over previous
import jax
import jax.numpy as jnp
from jax import lax
from jax.experimental import pallas as pl
from jax.experimental.pallas import tpu as pltpu
from jax.experimental.pallas import tpu_sc as plsc

VOCAB = 1000000
EMBED_DIM = 64
BATCH = 4096
HIST_LEN = 200

NC = 2
NS = 16
NW = NC * NS
L = 16

NPAIR = VOCAB // 2
NBUF = 4
SBUF = 2


PITCH = 65


def _gather_body(idx_hbm, tbl_hbm, out_hbm, slab_v, ibuf_v, pofs_v, g_v,
                 p65_v, sel_v, gsems, ssems):
    w = lax.axis_index("s") * NC + lax.axis_index("c")
    iota = lax.iota(jnp.int32, L)
    l65 = [(iota + L * m) * PITCH for m in range(8)]

    pltpu.sync_copy(idx_hbm.at[:, pl.ds(128 * w, 128)], slab_v)

    def build(t, b):
        for m in range(8):
            v = slab_v[t, pl.ds(L * m, L)]
            ibuf_v[b, pl.ds(L * m, L)] = lax.shift_right_logical(v, 1)
            pofs_v[b, pl.ds(L * m, L)] = lax.shift_left(
                lax.bitwise_and(v, 1), 6)

    def fire(t, b):
        build(t, b)
        pltpu.async_copy(tbl_hbm.at[ibuf_v.at[b]], g_v.at[b], gsems.at[b])

    def gwait(b):
        pltpu.make_async_copy(tbl_hbm.at[ibuf_v.at[b]], g_v.at[b],
                              gsems.at[b]).wait()

    def select(b, s):
        def row(l, _):
            lsp = lax.broadcast(l, (L,))
            rbase = plsc.load_gather(pofs_v.at[b], [lsp]) + iota
            base = PITCH * l
            for c in range(4):
                p65_v[pl.ds(base + L * c, L)] = plsc.load_gather(
                    g_v.at[b, l], [rbase + L * c])
            return ()
        lax.fori_loop(0, 128, row, (), unroll=8)

        def drow(d, _):
            for m in range(8):
                sel_v[s, d, pl.ds(L * m, L)] = plsc.load_gather(
                    p65_v, [l65[m] + d])
            return ()
        lax.fori_loop(0, EMBED_DIM, drow, (), unroll=4)

    def store(t, s):
        pltpu.async_copy(sel_v.at[s],
                         out_hbm.at[t, :, pl.ds(128 * w, 128)], ssems.at[s])

    def swait(t, s):
        pltpu.make_async_copy(sel_v.at[s],
                              out_hbm.at[t, :, pl.ds(128 * w, 128)],
                              ssems.at[s]).wait()

    def step(t, b, s, prev_store, next_fire):
        gwait(b)
        if prev_store:
            swait(t - SBUF, s)
        select(b, s)
        store(t, s)
        if next_fire:
            fire(t + NBUF, b)

    for b in range(NBUF):
        fire(b, b)
    for t in range(NBUF):
        step(t, t, t % SBUF, t >= SBUF, True)

    def body(gi, _):
        for b in range(NBUF):
            step(NBUF * gi + b, b, b % SBUF, True, True)
        return ()

    lax.fori_loop(1, HIST_LEN // NBUF - 1, body, (), unroll=False)

    for b in range(NBUF):
        step(HIST_LEN - NBUF + b, b, b % SBUF, True, False)
    for s in range(SBUF):
        swait(HIST_LEN - SBUF + s, s)


@jax.jit
def _gather(idx_t, tbl):
    mesh = plsc.VectorSubcoreMesh(core_axis_name="c", subcore_axis_name="s")
    kern = pl.kernel(
        _gather_body,
        out_type=jax.ShapeDtypeStruct((HIST_LEN, EMBED_DIM, BATCH),
                                      jnp.float32),
        mesh=mesh,
        scratch_types=[
            pltpu.VMEM((HIST_LEN, 128), jnp.int32),
            pltpu.VMEM((NBUF, 128), jnp.int32),
            pltpu.VMEM((NBUF, 128), jnp.int32),
            pltpu.VMEM((NBUF, 128, 128), jnp.float32),
            pltpu.VMEM((PITCH * 128,), jnp.float32),
            pltpu.VMEM((SBUF, EMBED_DIM, 128), jnp.float32),
            pltpu.SemaphoreType.DMA((NBUF,)),
            pltpu.SemaphoreType.DMA((SBUF,)),
        ],
        compiler_params=pltpu.CompilerParams(
            use_tc_tiling_on_sc=True, needs_layout_passes=False),
    )
    return kern(idx_t, tbl)


def kernel(inputs, embeddings):
    idx_t = inputs.astype(jnp.int32).T
    tbl = embeddings.reshape(NPAIR, 128)
    out_t = _gather(idx_t, tbl)
    return out_t.transpose(2, 0, 1)
